# baseline (speedup 1.0000x reference)
import functools

import numpy as np
import jax
import jax.numpy as jnp
from jax import lax
from jax.experimental import pallas as pl
from jax.experimental.pallas import tpu as pltpu
from jax.experimental.pallas import tpu_sc as plsc

NC = 2
NS = 16
NW = NC * NS
LANES = 16
EB = 128
CW = 16
ROW_BLOCK = 5000


def _cdiv(a, b):
    return (a + b - 1) // b


def _mesh():
    return plsc.VectorSubcoreMesh(core_axis_name="c", subcore_axis_name="s")


_SC_PARAMS = pltpu.CompilerParams(use_tc_tiling_on_sc=False)


def _stage_idx(raw_hbm, pad_hbm, idx_v, wid, s_steps, e_rows):
    last = NW - 1
    r_real = e_rows - last * s_steps
    r_pad = s_steps - r_real

    @pl.when(wid < last)
    def _():
        pltpu.sync_copy(raw_hbm.at[pl.ds(wid * s_steps, s_steps)], idx_v)

    @pl.when(wid == last)
    def _():
        pltpu.sync_copy(raw_hbm.at[pl.ds(last * s_steps, r_real)],
                        idx_v.at[pl.ds(0, r_real)])
        pltpu.sync_copy(pad_hbm, idx_v.at[pl.ds(r_real, r_pad)])


def _make_degree_kernel(n_pad, s_steps, e_rows):
    rpt = n_pad // NS

    @functools.partial(
        pl.kernel,
        out_type=jax.ShapeDtypeStruct((NC, n_pad, CW), jnp.float32),
        mesh=_mesh(),
        compiler_params=_SC_PARAMS,
        scratch_types=[
            pltpu.VMEM((s_steps, EB), jnp.int32),
            pltpu.VMEM((EB, CW), jnp.float32),
            pltpu.VMEM((rpt, CW), jnp.float32),
            pltpu.VMEM_SHARED((n_pad, CW), jnp.float32),
            pltpu.SemaphoreType.DMA,
        ],
    )
    def degree(dst_hbm, pad_hbm, out_hbm, dst_v, ones_v, bounce_v, acc_sh, sem):
        c = lax.axis_index("c")
        s = lax.axis_index("s")
        wid = c * NS + s
        _stage_idx(dst_hbm, pad_hbm, dst_v, wid, s_steps, e_rows)

        one = jnp.ones((LANES,), jnp.float32)
        zero = jnp.zeros((LANES,), jnp.float32)

        def fill_ones(i, _):
            ones_v[i, pl.ds(0, LANES)] = one
            return 0

        lax.fori_loop(0, EB, fill_ones, 0)

        def fill_zero(i, _):
            bounce_v[i, pl.ds(0, LANES)] = zero
            return 0

        lax.fori_loop(0, rpt, fill_zero, 0)
        pltpu.sync_copy(bounce_v, acc_sh.at[pl.ds(s * rpt, rpt)])
        plsc.subcore_barrier()

        K = 8

        def step(j2, _):
            cps = [pltpu.async_copy(ones_v, acc_sh.at[dst_v.at[j2 * K + b]],
                                    sem, add=True) for b in range(K)]
            for cp in cps:
                cp.wait()
            return 0

        lax.fori_loop(0, s_steps // K, step, 0)
        plsc.subcore_barrier()

        pltpu.sync_copy(acc_sh.at[pl.ds(s * rpt, rpt)], bounce_v)
        pltpu.sync_copy(bounce_v, out_hbm.at[c, pl.ds(s * rpt, rpt)])

    return degree


def _make_propagate_kernel(n_pad, width, s_steps, e_rows):
    rpt = n_pad // NS
    nb = 8
    assert s_steps % nb == 0

    @functools.partial(
        pl.kernel,
        out_type=jax.ShapeDtypeStruct((NC, n_pad, width), jnp.float32),
        mesh=_mesh(),
        compiler_params=_SC_PARAMS,
        scratch_types=[
            pltpu.VMEM((s_steps, EB), jnp.int32),
            pltpu.VMEM((s_steps, EB), jnp.int32),
            pltpu.VMEM((nb, EB, width), jnp.float32),
            pltpu.VMEM((rpt, width), jnp.float32),
            pltpu.VMEM_SHARED((n_pad, width), jnp.float32),
        ] + [pltpu.SemaphoreType.DMA] * (2 * nb),
    )
    def propagate(u_hbm, src_hbm, dst_hbm, psrc_hbm, pdst_hbm, out_hbm,
                  src_v, dst_v, rows_v, bounce_v, acc_sh, *sems):
        gsems = sems[:nb]
        ssems = sems[nb:]
        c = lax.axis_index("c")
        s = lax.axis_index("s")
        wid = c * NS + s
        _stage_idx(src_hbm, psrc_hbm, src_v, wid, s_steps, e_rows)
        _stage_idx(dst_hbm, pdst_hbm, dst_v, wid, s_steps, e_rows)

        zero = jnp.zeros((LANES,), jnp.float32)

        def fill_zero(i, _):
            for k in range(width // LANES):
                bounce_v[i, pl.ds(k * LANES, LANES)] = zero
            return 0

        lax.fori_loop(0, rpt, fill_zero, 0)
        pltpu.sync_copy(bounce_v, acc_sh.at[pl.ds(s * rpt, rpt)])
        plsc.subcore_barrier()

        for b in range(nb):
            pltpu.async_copy(u_hbm.at[src_v.at[b]], rows_v.at[b], gsems[b])

        def blk(j2, _):
            base = j2 * nb
            for b in range(nb):
                j = base + b
                pltpu.make_async_copy(
                    u_hbm.at[src_v.at[j]], rows_v.at[b], gsems[b]).wait()
                pltpu.async_copy(rows_v.at[b], acc_sh.at[dst_v.at[j]],
                                 ssems[b], add=True)
            for b in range(nb):
                nj = base + nb + b

                @pl.when(nj < s_steps)
                def _():
                    pltpu.make_async_copy(
                        rows_v.at[b], acc_sh.at[dst_v.at[base + b]],
                        ssems[b]).wait()
                    pltpu.async_copy(
                        u_hbm.at[src_v.at[nj]], rows_v.at[b], gsems[b])
            return 0

        lax.fori_loop(0, s_steps // nb, blk, 0)
        for b in range(nb):
            pltpu.make_async_copy(
                rows_v.at[b], acc_sh.at[dst_v.at[s_steps - nb + b]],
                ssems[b]).wait()
        plsc.subcore_barrier()

        pltpu.sync_copy(acc_sh.at[pl.ds(s * rpt, rpt)], bounce_v)
        pltpu.sync_copy(bounce_v, out_hbm.at[c, pl.ds(s * rpt, rpt)])

    return propagate


def _dinv_block(cnt_ref):
    deg = cnt_ref[0, :, 0:1] + cnt_ref[1, :, 0:1] + 1.0
    return lax.rsqrt(deg)


def _t1_body(x_ref, w1_ref, cnt_ref, u1_ref):
    dinv = _dinv_block(cnt_ref)
    t = jnp.dot(x_ref[...], w1_ref[...], preferred_element_type=jnp.float32,
                precision=lax.Precision.HIGHEST)
    u1_ref[...] = t * dinv


def _t2_body(p_ref, u1_ref, cnt_ref, b1_ref, u2_ref):
    dinv = _dinv_block(cnt_ref)
    sfull = (p_ref[0] + p_ref[1] + u1_ref[...]) * dinv
    h = jnp.maximum(sfull + b1_ref[...], 0.0)
    u2_ref[...] = h * dinv


def _t3_body(q_ref, u2_ref, cnt_ref,
             wmu_ref, bmu_ref, wvar_ref, bvar_ref, mu_ref, sg_ref):
    dinv = _dinv_block(cnt_ref)
    z = (q_ref[0] + q_ref[1] + u2_ref[...]) * dinv
    mu_ref[...] = jnp.dot(z, wmu_ref[...], preferred_element_type=jnp.float32,
                          precision=lax.Precision.HIGHEST) + bmu_ref[...]
    sg_ref[...] = jnp.dot(z, wvar_ref[...], preferred_element_type=jnp.float32,
                          precision=lax.Precision.HIGHEST) + bvar_ref[...]


def kernel(x, edge_index, w1, b1, w_mu, b_mu, w_var, b_var):
    n, d_in = x.shape
    h_dim = w1.shape[1]
    d_out = w_mu.shape[1]
    e = edge_index.shape[1]

    n_pad = _cdiv(n + 1, NS * 8) * NS * 8
    e_rows = e // EB
    s_steps = _cdiv(_cdiv(e, NW * EB), 8) * 8
    pad_rows = s_steps - (e_rows - (NW - 1) * s_steps)

    pk = np.arange(pad_rows * EB, dtype=np.int32)
    pad_src = jnp.asarray((pk % n).reshape(pad_rows, EB))
    pad_dst = jnp.asarray((n + pk % (n_pad - n)).reshape(pad_rows, EB))

    src2d = edge_index[0].reshape(e_rows, EB)
    dst2d = edge_index[1].reshape(e_rows, EB)

    grid = (n // ROW_BLOCK,)
    row2 = lambda i: (i, 0)
    row3 = lambda i: (0, i, 0)
    full2 = lambda i: (0, 0)
    cnt_spec = pl.BlockSpec((NC, ROW_BLOCK, CW), row3)

    cnt = _make_degree_kernel(n_pad, s_steps, e_rows)(dst2d, pad_dst)

    u1 = pl.pallas_call(
        _t1_body,
        grid=grid,
        in_specs=[
            pl.BlockSpec((ROW_BLOCK, d_in), row2),
            pl.BlockSpec((d_in, h_dim), full2),
            cnt_spec,
        ],
        out_specs=pl.BlockSpec((ROW_BLOCK, h_dim), row2),
        out_shape=jax.ShapeDtypeStruct((n_pad, h_dim), jnp.float32),
    )(x, w1, cnt)

    prop = _make_propagate_kernel(n_pad, h_dim, s_steps, e_rows)
    p = prop(u1, src2d, dst2d, pad_src, pad_dst)

    u2 = pl.pallas_call(
        _t2_body,
        grid=grid,
        in_specs=[
            pl.BlockSpec((NC, ROW_BLOCK, h_dim), row3),
            pl.BlockSpec((ROW_BLOCK, h_dim), row2),
            cnt_spec,
            pl.BlockSpec((1, h_dim), full2),
        ],
        out_specs=pl.BlockSpec((ROW_BLOCK, h_dim), row2),
        out_shape=jax.ShapeDtypeStruct((n_pad, h_dim), jnp.float32),
    )(p, u1, cnt, b1.reshape(1, h_dim))

    q = prop(u2, src2d, dst2d, pad_src, pad_dst)

    mu, sg = pl.pallas_call(
        _t3_body,
        grid=grid,
        in_specs=[
            pl.BlockSpec((NC, ROW_BLOCK, h_dim), row3),
            pl.BlockSpec((ROW_BLOCK, h_dim), row2),
            cnt_spec,
            pl.BlockSpec((h_dim, d_out), full2),
            pl.BlockSpec((1, d_out), full2),
            pl.BlockSpec((h_dim, d_out), full2),
            pl.BlockSpec((1, d_out), full2),
        ],
        out_specs=[
            pl.BlockSpec((ROW_BLOCK, d_out), row2),
            pl.BlockSpec((ROW_BLOCK, d_out), row2),
        ],
        out_shape=[
            jax.ShapeDtypeStruct((n, d_out), jnp.float32),
            jax.ShapeDtypeStruct((n, d_out), jnp.float32),
        ],
    )(q, u2, cnt, w_mu, b_mu.reshape(1, d_out), w_var, b_var.reshape(1, d_out))

    return (mu, sg)


# speedup vs baseline: 1.0231x; 1.0231x over previous
import functools

import numpy as np
import jax
import jax.numpy as jnp
from jax import lax
from jax.experimental import pallas as pl
from jax.experimental.pallas import tpu as pltpu
from jax.experimental.pallas import tpu_sc as plsc

NC = 2
NS = 16
NW = NC * NS
LANES = 16
EB = 128
CW = 16
ROW_BLOCK = 2000


def _cdiv(a, b):
    return (a + b - 1) // b


def _mesh():
    return plsc.VectorSubcoreMesh(core_axis_name="c", subcore_axis_name="s")


_SC_PARAMS = pltpu.CompilerParams(use_tc_tiling_on_sc=False)


def _stage_idx(raw_hbm, pad_hbm, idx_v, wid, s_steps, e_rows):
    last = NW - 1
    r_real = e_rows - last * s_steps
    r_pad = s_steps - r_real

    @pl.when(wid < last)
    def _():
        pltpu.sync_copy(raw_hbm.at[pl.ds(wid * s_steps, s_steps)], idx_v)

    @pl.when(wid == last)
    def _():
        pltpu.sync_copy(raw_hbm.at[pl.ds(last * s_steps, r_real)],
                        idx_v.at[pl.ds(0, r_real)])
        pltpu.sync_copy(pad_hbm, idx_v.at[pl.ds(r_real, r_pad)])


def _make_degree_kernel(n_pad, s_steps, e_rows):
    rpt = n_pad // NS

    @functools.partial(
        pl.kernel,
        out_type=jax.ShapeDtypeStruct((NC, n_pad, CW), jnp.float32),
        mesh=_mesh(),
        compiler_params=_SC_PARAMS,
        scratch_types=[
            pltpu.VMEM((s_steps, EB), jnp.int32),
            pltpu.VMEM((EB, CW), jnp.float32),
            pltpu.VMEM((rpt, CW), jnp.float32),
            pltpu.VMEM_SHARED((n_pad, CW), jnp.float32),
            pltpu.SemaphoreType.DMA,
        ],
    )
    def degree(dst_hbm, pad_hbm, out_hbm, dst_v, ones_v, bounce_v, acc_sh, sem):
        c = lax.axis_index("c")
        s = lax.axis_index("s")
        wid = c * NS + s
        _stage_idx(dst_hbm, pad_hbm, dst_v, wid, s_steps, e_rows)

        one = jnp.ones((LANES,), jnp.float32)
        zero = jnp.zeros((LANES,), jnp.float32)

        def fill_ones(i, _):
            ones_v[i, pl.ds(0, LANES)] = one
            return 0

        lax.fori_loop(0, EB, fill_ones, 0)

        def fill_zero(i, _):
            bounce_v[i, pl.ds(0, LANES)] = zero
            return 0

        lax.fori_loop(0, rpt, fill_zero, 0)
        pltpu.sync_copy(bounce_v, acc_sh.at[pl.ds(s * rpt, rpt)])
        plsc.subcore_barrier()

        K = 8

        def step(j2, _):
            cps = [pltpu.async_copy(ones_v, acc_sh.at[dst_v.at[j2 * K + b]],
                                    sem, add=True) for b in range(K)]
            for cp in cps:
                cp.wait()
            return 0

        lax.fori_loop(0, s_steps // K, step, 0)
        plsc.subcore_barrier()

        pltpu.sync_copy(acc_sh.at[pl.ds(s * rpt, rpt)], bounce_v)
        pltpu.sync_copy(bounce_v, out_hbm.at[c, pl.ds(s * rpt, rpt)])

    return degree


def _make_fused_l2_kernel(n_pad, width, s_steps, e_rows):
    rpt = n_pad // NS
    nb = 8
    ch = rpt // 2
    assert s_steps % nb == 0 and rpt % 2 == 0

    @functools.partial(
        pl.kernel,
        out_type=[
            jax.ShapeDtypeStruct((NC, n_pad, width), jnp.float32),
            jax.ShapeDtypeStruct((NC * n_pad, width), jnp.float32),
        ],
        mesh=_mesh(),
        compiler_params=_SC_PARAMS,
        scratch_types=[
            pltpu.VMEM((s_steps, EB), jnp.int32),
            pltpu.VMEM((s_steps, EB), jnp.int32),
            pltpu.VMEM((nb, EB, width), jnp.float32),
            pltpu.VMEM((rpt, width), jnp.float32),
            pltpu.VMEM((ch, width), jnp.float32),
            pltpu.VMEM((ch, width), jnp.float32),
            pltpu.VMEM((width,), jnp.float32),
            pltpu.VMEM_SHARED((n_pad, width), jnp.float32),
        ] + [pltpu.SemaphoreType.DMA] * (2 * nb),
    )
    def fused(p_hbm, u1_hbm, db_hbm, b1_hbm, src_hbm, dst_hbm,
              psrc_hbm, pdst_hbm, q_hbm, u2_hbm,
              src_v, dst_v, rows_v, bounce_v, ew_a, ew_b, b1_v, acc_sh, *sems):
        gsems = sems[:nb]
        ssems = sems[nb:]
        c = lax.axis_index("c")
        s = lax.axis_index("s")
        wid = c * NS + s
        _stage_idx(src_hbm, psrc_hbm, src_v, wid, s_steps, e_rows)
        _stage_idx(dst_hbm, pdst_hbm, dst_v, wid, s_steps, e_rows)
        pltpu.sync_copy(b1_hbm, b1_v)

        coff = jnp.broadcast_to((c * n_pad).astype(jnp.int32), (LANES,))

        def bias_idx(r, _):
            for k in range(EB // LANES):
                sl = pl.ds(k * LANES, LANES)
                src_v[r, sl] = src_v[r, sl] + coff
            return 0

        lax.fori_loop(0, s_steps, bias_idx, 0)

        zero = jnp.zeros((LANES,), jnp.float32)
        nh = width // LANES

        def fill_zero(i, _):
            for k in range(nh):
                bounce_v[i, pl.ds(k * LANES, LANES)] = zero
            return 0

        lax.fori_loop(0, rpt, fill_zero, 0)
        pltpu.sync_copy(bounce_v, acc_sh.at[pl.ds(s * rpt, rpt)])

        b1vec = [b1_v[pl.ds(k * LANES, LANES)] for k in range(nh)]

        def ew_add(r, _):
            for k in range(nh):
                sl = pl.ds(k * LANES, LANES)
                ew_a[r, sl] = ew_a[r, sl] + ew_b[r, sl]
            return 0

        def ew_fin(r, _):
            for k in range(nh):
                sl = pl.ds(k * LANES, LANES)
                a = ew_a[r, sl] * ew_b[r, sl] + b1vec[k]
                ew_a[r, sl] = jnp.maximum(a, 0.0) * ew_b[r, sl]
            return 0

        for half in range(2):
            row0 = s * rpt + half * ch
            sl = pl.ds(row0, ch)
            pltpu.sync_copy(p_hbm.at[0, sl], ew_a)
            pltpu.sync_copy(p_hbm.at[1, sl], ew_b)
            lax.fori_loop(0, ch, ew_add, 0)
            pltpu.sync_copy(u1_hbm.at[sl], ew_b)
            lax.fori_loop(0, ch, ew_add, 0)
            pltpu.sync_copy(db_hbm.at[sl], ew_b)
            lax.fori_loop(0, ch, ew_fin, 0)
            pltpu.sync_copy(ew_a, u2_hbm.at[pl.ds(c * n_pad + row0, ch)])
        plsc.subcore_barrier()

        for b in range(nb):
            pltpu.async_copy(u2_hbm.at[src_v.at[b]], rows_v.at[b], gsems[b])

        def blk(j2, _):
            base = j2 * nb
            for b in range(nb):
                j = base + b
                pltpu.make_async_copy(
                    u2_hbm.at[src_v.at[j]], rows_v.at[b], gsems[b]).wait()
                pltpu.async_copy(rows_v.at[b], acc_sh.at[dst_v.at[j]],
                                 ssems[b], add=True)
            for b in range(nb):
                nj = base + nb + b

                @pl.when(nj < s_steps)
                def _():
                    pltpu.make_async_copy(
                        rows_v.at[b], acc_sh.at[dst_v.at[base + b]],
                        ssems[b]).wait()
                    pltpu.async_copy(
                        u2_hbm.at[src_v.at[nj]], rows_v.at[b], gsems[b])
            return 0

        lax.fori_loop(0, s_steps // nb, blk, 0)
        for b in range(nb):
            pltpu.make_async_copy(
                rows_v.at[b], acc_sh.at[dst_v.at[s_steps - nb + b]],
                ssems[b]).wait()
        plsc.subcore_barrier()

        pltpu.sync_copy(acc_sh.at[pl.ds(s * rpt, rpt)], bounce_v)
        pltpu.sync_copy(bounce_v, q_hbm.at[c, pl.ds(s * rpt, rpt)])

    return fused


def _make_propagate_kernel(n_pad, width, s_steps, e_rows):
    rpt = n_pad // NS
    nb = 8
    assert s_steps % nb == 0

    @functools.partial(
        pl.kernel,
        out_type=jax.ShapeDtypeStruct((NC, n_pad, width), jnp.float32),
        mesh=_mesh(),
        compiler_params=_SC_PARAMS,
        scratch_types=[
            pltpu.VMEM((s_steps, EB), jnp.int32),
            pltpu.VMEM((s_steps, EB), jnp.int32),
            pltpu.VMEM((nb, EB, width), jnp.float32),
            pltpu.VMEM((rpt, width), jnp.float32),
            pltpu.VMEM_SHARED((n_pad, width), jnp.float32),
        ] + [pltpu.SemaphoreType.DMA] * (2 * nb),
    )
    def propagate(u_hbm, src_hbm, dst_hbm, psrc_hbm, pdst_hbm, out_hbm,
                  src_v, dst_v, rows_v, bounce_v, acc_sh, *sems):
        gsems = sems[:nb]
        ssems = sems[nb:]
        c = lax.axis_index("c")
        s = lax.axis_index("s")
        wid = c * NS + s
        _stage_idx(src_hbm, psrc_hbm, src_v, wid, s_steps, e_rows)
        _stage_idx(dst_hbm, pdst_hbm, dst_v, wid, s_steps, e_rows)

        zero = jnp.zeros((LANES,), jnp.float32)

        def fill_zero(i, _):
            for k in range(width // LANES):
                bounce_v[i, pl.ds(k * LANES, LANES)] = zero
            return 0

        lax.fori_loop(0, rpt, fill_zero, 0)
        pltpu.sync_copy(bounce_v, acc_sh.at[pl.ds(s * rpt, rpt)])
        plsc.subcore_barrier()

        for b in range(nb):
            pltpu.async_copy(u_hbm.at[src_v.at[b]], rows_v.at[b], gsems[b])

        def blk(j2, _):
            base = j2 * nb
            for b in range(nb):
                j = base + b
                pltpu.make_async_copy(
                    u_hbm.at[src_v.at[j]], rows_v.at[b], gsems[b]).wait()
                pltpu.async_copy(rows_v.at[b], acc_sh.at[dst_v.at[j]],
                                 ssems[b], add=True)
            for b in range(nb):
                nj = base + nb + b

                @pl.when(nj < s_steps)
                def _():
                    pltpu.make_async_copy(
                        rows_v.at[b], acc_sh.at[dst_v.at[base + b]],
                        ssems[b]).wait()
                    pltpu.async_copy(
                        u_hbm.at[src_v.at[nj]], rows_v.at[b], gsems[b])
            return 0

        lax.fori_loop(0, s_steps // nb, blk, 0)
        for b in range(nb):
            pltpu.make_async_copy(
                rows_v.at[b], acc_sh.at[dst_v.at[s_steps - nb + b]],
                ssems[b]).wait()
        plsc.subcore_barrier()

        pltpu.sync_copy(acc_sh.at[pl.ds(s * rpt, rpt)], bounce_v)
        pltpu.sync_copy(bounce_v, out_hbm.at[c, pl.ds(s * rpt, rpt)])

    return propagate


def _dinv_block(cnt_ref):
    deg = cnt_ref[0, :, 0:1] + cnt_ref[1, :, 0:1] + 1.0
    return lax.rsqrt(deg)


def _t1_body(x_ref, w1_ref, cnt_ref, u1_ref, db_ref):
    dinv = _dinv_block(cnt_ref)
    t = jnp.dot(x_ref[...], w1_ref[...], preferred_element_type=jnp.float32,
                precision=lax.Precision.HIGHEST)
    u1_ref[...] = t * dinv
    db_ref[...] = jnp.broadcast_to(dinv, db_ref.shape)


def _t3_body(q_ref, u2_ref, db_ref,
             wmu_ref, bmu_ref, wvar_ref, bvar_ref, mu_ref, sg_ref):
    z = (q_ref[0] + q_ref[1] + u2_ref[...]) * db_ref[...]
    mu_ref[...] = jnp.dot(z, wmu_ref[...], preferred_element_type=jnp.float32,
                          precision=lax.Precision.HIGHEST) + bmu_ref[...]
    sg_ref[...] = jnp.dot(z, wvar_ref[...], preferred_element_type=jnp.float32,
                          precision=lax.Precision.HIGHEST) + bvar_ref[...]


def kernel(x, edge_index, w1, b1, w_mu, b_mu, w_var, b_var):
    n, d_in = x.shape
    h_dim = w1.shape[1]
    d_out = w_mu.shape[1]
    e = edge_index.shape[1]

    n_pad = _cdiv(n + 1, NS * 8) * NS * 8
    e_rows = e // EB
    s_steps = _cdiv(_cdiv(e, NW * EB), 8) * 8
    pad_rows = s_steps - (e_rows - (NW - 1) * s_steps)

    pk = np.arange(pad_rows * EB, dtype=np.int32)
    pad_src = jnp.asarray((pk % n).reshape(pad_rows, EB))
    pad_dst = jnp.asarray((n + pk % (n_pad - n)).reshape(pad_rows, EB))

    src2d = edge_index[0].reshape(e_rows, EB)
    dst2d = edge_index[1].reshape(e_rows, EB)

    grid = (n // ROW_BLOCK,)
    row2 = lambda i: (i, 0)
    row3 = lambda i: (0, i, 0)
    full2 = lambda i: (0, 0)
    cnt_spec = pl.BlockSpec((NC, ROW_BLOCK, CW), row3)

    cnt = _make_degree_kernel(n_pad, s_steps, e_rows)(dst2d, pad_dst)

    u1, db = pl.pallas_call(
        _t1_body,
        grid=grid,
        in_specs=[
            pl.BlockSpec((ROW_BLOCK, d_in), row2),
            pl.BlockSpec((d_in, h_dim), full2),
            cnt_spec,
        ],
        out_specs=[
            pl.BlockSpec((ROW_BLOCK, h_dim), row2),
            pl.BlockSpec((ROW_BLOCK, h_dim), row2),
        ],
        out_shape=[
            jax.ShapeDtypeStruct((n_pad, h_dim), jnp.float32),
            jax.ShapeDtypeStruct((n_pad, h_dim), jnp.float32),
        ],
    )(x, w1, cnt)

    prop = _make_propagate_kernel(n_pad, h_dim, s_steps, e_rows)
    p = prop(u1, src2d, dst2d, pad_src, pad_dst)

    q, u2 = _make_fused_l2_kernel(n_pad, h_dim, s_steps, e_rows)(
        p, u1, db, b1, src2d, dst2d, pad_src, pad_dst)

    mu, sg = pl.pallas_call(
        _t3_body,
        grid=grid,
        in_specs=[
            pl.BlockSpec((NC, ROW_BLOCK, h_dim), row3),
            pl.BlockSpec((ROW_BLOCK, h_dim), row2),
            pl.BlockSpec((ROW_BLOCK, h_dim), row2),
            pl.BlockSpec((h_dim, d_out), full2),
            pl.BlockSpec((1, d_out), full2),
            pl.BlockSpec((h_dim, d_out), full2),
            pl.BlockSpec((1, d_out), full2),
        ],
        out_specs=[
            pl.BlockSpec((ROW_BLOCK, d_out), row2),
            pl.BlockSpec((ROW_BLOCK, d_out), row2),
        ],
        out_shape=[
            jax.ShapeDtypeStruct((n, d_out), jnp.float32),
            jax.ShapeDtypeStruct((n, d_out), jnp.float32),
        ],
    )(q, u2, db, w_mu, b_mu.reshape(1, d_out), w_var, b_var.reshape(1, d_out))

    return (mu, sg)


# speedup vs baseline: 1.0515x; 1.0278x over previous
import functools

import numpy as np
import jax
import jax.numpy as jnp
from jax import lax
from jax.experimental import pallas as pl
from jax.experimental.pallas import tpu as pltpu
from jax.experimental.pallas import tpu_sc as plsc

NC = 2
NS = 16
NW = NC * NS
LANES = 16
EB = 128
CW = 16
ROW_BLOCK = 2000


def _cdiv(a, b):
    return (a + b - 1) // b


def _mesh():
    return plsc.VectorSubcoreMesh(core_axis_name="c", subcore_axis_name="s")


_SC_PARAMS = pltpu.CompilerParams(use_tc_tiling_on_sc=False)


def _stage_idx(raw_hbm, pad_hbm, idx_v, wid, s_steps, e_rows):
    last = NW - 1
    r_real = e_rows - last * s_steps
    r_pad = s_steps - r_real

    @pl.when(wid < last)
    def _():
        pltpu.sync_copy(raw_hbm.at[pl.ds(wid * s_steps, s_steps)], idx_v)

    @pl.when(wid == last)
    def _():
        pltpu.sync_copy(raw_hbm.at[pl.ds(last * s_steps, r_real)],
                        idx_v.at[pl.ds(0, r_real)])
        pltpu.sync_copy(pad_hbm, idx_v.at[pl.ds(r_real, r_pad)])


def _make_degree_kernel(n_pad, s_steps, e_rows):
    rpt = n_pad // NS

    @functools.partial(
        pl.kernel,
        out_type=jax.ShapeDtypeStruct((NC, n_pad, CW), jnp.float32),
        mesh=_mesh(),
        compiler_params=_SC_PARAMS,
        scratch_types=[
            pltpu.VMEM((s_steps, EB), jnp.int32),
            pltpu.VMEM((EB, CW), jnp.float32),
            pltpu.VMEM((rpt, CW), jnp.float32),
            pltpu.VMEM_SHARED((n_pad, CW), jnp.float32),
            pltpu.SemaphoreType.DMA,
        ],
    )
    def degree(dst_hbm, pad_hbm, out_hbm, dst_v, ones_v, bounce_v, acc_sh, sem):
        c = lax.axis_index("c")
        s = lax.axis_index("s")
        wid = c * NS + s
        _stage_idx(dst_hbm, pad_hbm, dst_v, wid, s_steps, e_rows)

        one = jnp.ones((LANES,), jnp.float32)
        zero = jnp.zeros((LANES,), jnp.float32)

        def fill_ones(i, _):
            ones_v[i, pl.ds(0, LANES)] = one
            return 0

        lax.fori_loop(0, EB, fill_ones, 0)

        def fill_zero(i, _):
            bounce_v[i, pl.ds(0, LANES)] = zero
            return 0

        lax.fori_loop(0, rpt, fill_zero, 0)
        pltpu.sync_copy(bounce_v, acc_sh.at[pl.ds(s * rpt, rpt)])
        plsc.subcore_barrier()

        K = 8

        def step(j2, _):
            cps = [pltpu.async_copy(ones_v, acc_sh.at[dst_v.at[j2 * K + b]],
                                    sem, add=True) for b in range(K)]
            for cp in cps:
                cp.wait()
            return 0

        lax.fori_loop(0, s_steps // K, step, 0)
        plsc.subcore_barrier()

        pltpu.sync_copy(acc_sh.at[pl.ds(s * rpt, rpt)], bounce_v)
        pltpu.sync_copy(bounce_v, out_hbm.at[c, pl.ds(s * rpt, rpt)])

    return degree


def _make_propagate_kernel(n_pad, width, s_steps, e_rows):
    rpt = n_pad // NS
    nb = 8
    assert s_steps % nb == 0

    @functools.partial(
        pl.kernel,
        out_type=jax.ShapeDtypeStruct((NC, n_pad, width), jnp.float32),
        mesh=_mesh(),
        compiler_params=_SC_PARAMS,
        scratch_types=[
            pltpu.VMEM((s_steps, EB), jnp.int32),
            pltpu.VMEM((s_steps, EB), jnp.int32),
            pltpu.VMEM((nb, EB, width), jnp.float32),
            pltpu.VMEM((rpt, width), jnp.float32),
            pltpu.VMEM_SHARED((n_pad, width), jnp.float32),
        ] + [pltpu.SemaphoreType.DMA] * (2 * nb),
    )
    def propagate(u_hbm, src_hbm, dst_hbm, psrc_hbm, pdst_hbm, out_hbm,
                  src_v, dst_v, rows_v, bounce_v, acc_sh, *sems):
        gsems = sems[:nb]
        ssems = sems[nb:]
        c = lax.axis_index("c")
        s = lax.axis_index("s")
        wid = c * NS + s
        _stage_idx(src_hbm, psrc_hbm, src_v, wid, s_steps, e_rows)
        _stage_idx(dst_hbm, pdst_hbm, dst_v, wid, s_steps, e_rows)

        zero = jnp.zeros((LANES,), jnp.float32)

        def fill_zero(i, _):
            for k in range(width // LANES):
                bounce_v[i, pl.ds(k * LANES, LANES)] = zero
            return 0

        lax.fori_loop(0, rpt, fill_zero, 0)
        pltpu.sync_copy(bounce_v, acc_sh.at[pl.ds(s * rpt, rpt)])
        plsc.subcore_barrier()

        for b in range(nb):
            pltpu.async_copy(u_hbm.at[src_v.at[b]], rows_v.at[b], gsems[b])

        def blk(j2, _):
            base = j2 * nb
            for b in range(nb):
                j = base + b
                pltpu.make_async_copy(
                    u_hbm.at[src_v.at[j]], rows_v.at[b], gsems[b]).wait()
                pltpu.async_copy(rows_v.at[b], acc_sh.at[dst_v.at[j]],
                                 ssems[b], add=True)
            for b in range(nb):
                nj = base + nb + b

                @pl.when(nj < s_steps)
                def _():
                    pltpu.make_async_copy(
                        rows_v.at[b], acc_sh.at[dst_v.at[base + b]],
                        ssems[b]).wait()
                    pltpu.async_copy(
                        u_hbm.at[src_v.at[nj]], rows_v.at[b], gsems[b])
            return 0

        lax.fori_loop(0, s_steps // nb, blk, 0)
        for b in range(nb):
            pltpu.make_async_copy(
                rows_v.at[b], acc_sh.at[dst_v.at[s_steps - nb + b]],
                ssems[b]).wait()
        plsc.subcore_barrier()

        pltpu.sync_copy(acc_sh.at[pl.ds(s * rpt, rpt)], bounce_v)
        pltpu.sync_copy(bounce_v, out_hbm.at[c, pl.ds(s * rpt, rpt)])

    return propagate


def _dinv_block(cnt_ref):
    deg = cnt_ref[0, :, 0:1] + cnt_ref[1, :, 0:1] + 1.0
    return lax.rsqrt(deg)


def _t1_body(x_ref, w1_ref, cnt_ref, u1_ref):
    dinv = _dinv_block(cnt_ref)
    t = jnp.dot(x_ref[...], w1_ref[...], preferred_element_type=jnp.float32,
                precision=lax.Precision.HIGHEST)
    u1_ref[...] = t * dinv


def _t2_body(p_ref, u1_ref, cnt_ref, b1_ref, u2_ref):
    dinv = _dinv_block(cnt_ref)
    sfull = (p_ref[0] + p_ref[1] + u1_ref[...]) * dinv
    h = jnp.maximum(sfull + b1_ref[...], 0.0)
    u2_ref[...] = h * dinv


def _t3_body(q_ref, u2_ref, cnt_ref,
             wmu_ref, bmu_ref, wvar_ref, bvar_ref, mu_ref, sg_ref):
    dinv = _dinv_block(cnt_ref)
    z = (q_ref[0] + q_ref[1] + u2_ref[...]) * dinv
    mu_ref[...] = jnp.dot(z, wmu_ref[...], preferred_element_type=jnp.float32,
                          precision=lax.Precision.HIGHEST) + bmu_ref[...]
    sg_ref[...] = jnp.dot(z, wvar_ref[...], preferred_element_type=jnp.float32,
                          precision=lax.Precision.HIGHEST) + bvar_ref[...]


def kernel(x, edge_index, w1, b1, w_mu, b_mu, w_var, b_var):
    n, d_in = x.shape
    h_dim = w1.shape[1]
    d_out = w_mu.shape[1]
    e = edge_index.shape[1]

    n_pad = _cdiv(n + 1, NS * 8) * NS * 8
    e_rows = e // EB
    s_steps = _cdiv(_cdiv(e, NW * EB), 8) * 8
    pad_rows = s_steps - (e_rows - (NW - 1) * s_steps)

    pk = np.arange(pad_rows * EB, dtype=np.int32)
    pad_src = jnp.asarray((pk % n).reshape(pad_rows, EB))
    pad_dst = jnp.asarray((n + pk % (n_pad - n)).reshape(pad_rows, EB))

    src2d = edge_index[0].reshape(e_rows, EB)
    dst2d = edge_index[1].reshape(e_rows, EB)

    grid = (n // ROW_BLOCK,)
    row2 = lambda i: (i, 0)
    row3 = lambda i: (0, i, 0)
    full2 = lambda i: (0, 0)
    cnt_spec = pl.BlockSpec((NC, ROW_BLOCK, CW), row3)

    cnt = _make_degree_kernel(n_pad, s_steps, e_rows)(dst2d, pad_dst)

    u1 = pl.pallas_call(
        _t1_body,
        grid=grid,
        in_specs=[
            pl.BlockSpec((ROW_BLOCK, d_in), row2),
            pl.BlockSpec((d_in, h_dim), full2),
            cnt_spec,
        ],
        out_specs=pl.BlockSpec((ROW_BLOCK, h_dim), row2),
        out_shape=jax.ShapeDtypeStruct((n_pad, h_dim), jnp.float32),
    )(x, w1, cnt)

    prop = _make_propagate_kernel(n_pad, h_dim, s_steps, e_rows)
    p = prop(u1, src2d, dst2d, pad_src, pad_dst)

    u2 = pl.pallas_call(
        _t2_body,
        grid=grid,
        in_specs=[
            pl.BlockSpec((NC, ROW_BLOCK, h_dim), row3),
            pl.BlockSpec((ROW_BLOCK, h_dim), row2),
            cnt_spec,
            pl.BlockSpec((1, h_dim), full2),
        ],
        out_specs=pl.BlockSpec((ROW_BLOCK, h_dim), row2),
        out_shape=jax.ShapeDtypeStruct((n_pad, h_dim), jnp.float32),
    )(p, u1, cnt, b1.reshape(1, h_dim))

    q = prop(u2, src2d, dst2d, pad_src, pad_dst)

    mu, sg = pl.pallas_call(
        _t3_body,
        grid=grid,
        in_specs=[
            pl.BlockSpec((NC, ROW_BLOCK, h_dim), row3),
            pl.BlockSpec((ROW_BLOCK, h_dim), row2),
            cnt_spec,
            pl.BlockSpec((h_dim, d_out), full2),
            pl.BlockSpec((1, d_out), full2),
            pl.BlockSpec((h_dim, d_out), full2),
            pl.BlockSpec((1, d_out), full2),
        ],
        out_specs=[
            pl.BlockSpec((ROW_BLOCK, d_out), row2),
            pl.BlockSpec((ROW_BLOCK, d_out), row2),
        ],
        out_shape=[
            jax.ShapeDtypeStruct((n, d_out), jnp.float32),
            jax.ShapeDtypeStruct((n, d_out), jnp.float32),
        ],
    )(q, u2, cnt, w_mu, b_mu.reshape(1, d_out), w_var, b_var.reshape(1, d_out))

    return (mu, sg)


# speedup vs baseline: 1.0627x; 1.0107x over previous
import functools

import numpy as np
import jax
import jax.numpy as jnp
from jax import lax
from jax.experimental import pallas as pl
from jax.experimental.pallas import tpu as pltpu
from jax.experimental.pallas import tpu_sc as plsc

NC = 2
NS = 16
NW = NC * NS
LANES = 16
EB = 128
CW = 8
ROW_BLOCK = 2000


def _cdiv(a, b):
    return (a + b - 1) // b


def _mesh():
    return plsc.VectorSubcoreMesh(core_axis_name="c", subcore_axis_name="s")


_SC_PARAMS = pltpu.CompilerParams(use_tc_tiling_on_sc=False)


def _stage_idx(raw_hbm, pad_hbm, idx_v, wid, s_steps, e_rows):
    last = NW - 1
    r_real = e_rows - last * s_steps
    r_pad = s_steps - r_real

    @pl.when(wid < last)
    def _():
        pltpu.sync_copy(raw_hbm.at[pl.ds(wid * s_steps, s_steps)], idx_v)

    @pl.when(wid == last)
    def _():
        pltpu.sync_copy(raw_hbm.at[pl.ds(last * s_steps, r_real)],
                        idx_v.at[pl.ds(0, r_real)])
        pltpu.sync_copy(pad_hbm, idx_v.at[pl.ds(r_real, r_pad)])


def _make_degree_kernel(n_pad, s_steps, e_rows):
    rpt = n_pad // NS

    @functools.partial(
        pl.kernel,
        out_type=jax.ShapeDtypeStruct((NC, n_pad, CW), jnp.float32),
        mesh=_mesh(),
        compiler_params=_SC_PARAMS,
        scratch_types=[
            pltpu.VMEM((s_steps, EB), jnp.int32),
            pltpu.VMEM((EB, CW), jnp.float32),
            pltpu.VMEM((rpt, CW), jnp.float32),
            pltpu.VMEM_SHARED((n_pad, CW), jnp.float32),
            pltpu.SemaphoreType.DMA,
        ],
    )
    def degree(dst_hbm, pad_hbm, out_hbm, dst_v, ones_v, bounce_v, acc_sh, sem):
        c = lax.axis_index("c")
        s = lax.axis_index("s")
        wid = c * NS + s
        _stage_idx(dst_hbm, pad_hbm, dst_v, wid, s_steps, e_rows)

        one = jnp.ones((LANES,), jnp.float32)
        zero = jnp.zeros((LANES,), jnp.float32)

        def fill_ones(i, _):
            ones_v[i, pl.ds(0, LANES)] = one
            return 0

        lax.fori_loop(0, EB, fill_ones, 0)

        def fill_zero(i, _):
            bounce_v[i, pl.ds(0, LANES)] = zero
            return 0

        lax.fori_loop(0, rpt, fill_zero, 0)
        pltpu.sync_copy(bounce_v, acc_sh.at[pl.ds(s * rpt, rpt)])
        plsc.subcore_barrier()

        K = 8

        def step(j2, _):
            cps = [pltpu.async_copy(ones_v, acc_sh.at[dst_v.at[j2 * K + b]],
                                    sem, add=True) for b in range(K)]
            for cp in cps:
                cp.wait()
            return 0

        lax.fori_loop(0, s_steps // K, step, 0)
        plsc.subcore_barrier()

        pltpu.sync_copy(acc_sh.at[pl.ds(s * rpt, rpt)], bounce_v)
        pltpu.sync_copy(bounce_v, out_hbm.at[c, pl.ds(s * rpt, rpt)])

    return degree


def _make_propagate_kernel(n_pad, width, s_steps, e_rows):
    rpt = n_pad // NS
    nb = 8
    assert s_steps % nb == 0

    @functools.partial(
        pl.kernel,
        out_type=jax.ShapeDtypeStruct((NC, n_pad, width), jnp.float32),
        mesh=_mesh(),
        compiler_params=_SC_PARAMS,
        scratch_types=[
            pltpu.VMEM((s_steps, EB), jnp.int32),
            pltpu.VMEM((s_steps, EB), jnp.int32),
            pltpu.VMEM((nb, EB, width), jnp.float32),
            pltpu.VMEM((rpt, width), jnp.float32),
            pltpu.VMEM_SHARED((n_pad, width), jnp.float32),
        ] + [pltpu.SemaphoreType.DMA] * (2 * nb),
    )
    def propagate(u_hbm, src_hbm, dst_hbm, psrc_hbm, pdst_hbm, out_hbm,
                  src_v, dst_v, rows_v, bounce_v, acc_sh, *sems):
        gsems = sems[:nb]
        ssems = sems[nb:]
        c = lax.axis_index("c")
        s = lax.axis_index("s")
        wid = c * NS + s
        _stage_idx(src_hbm, psrc_hbm, src_v, wid, s_steps, e_rows)
        _stage_idx(dst_hbm, pdst_hbm, dst_v, wid, s_steps, e_rows)

        zero = jnp.zeros((LANES,), jnp.float32)

        def fill_zero(i, _):
            for k in range(width // LANES):
                bounce_v[i, pl.ds(k * LANES, LANES)] = zero
            return 0

        lax.fori_loop(0, rpt, fill_zero, 0)
        pltpu.sync_copy(bounce_v, acc_sh.at[pl.ds(s * rpt, rpt)])
        plsc.subcore_barrier()

        for b in range(nb):
            pltpu.async_copy(u_hbm.at[src_v.at[b]], rows_v.at[b], gsems[b])

        def blk(j2, _):
            base = j2 * nb
            for b in range(nb):
                j = base + b
                pltpu.make_async_copy(
                    u_hbm.at[src_v.at[j]], rows_v.at[b], gsems[b]).wait()
                pltpu.async_copy(rows_v.at[b], acc_sh.at[dst_v.at[j]],
                                 ssems[b], add=True)
            for b in range(nb):
                nj = base + nb + b

                @pl.when(nj < s_steps)
                def _():
                    pltpu.make_async_copy(
                        rows_v.at[b], acc_sh.at[dst_v.at[base + b]],
                        ssems[b]).wait()
                    pltpu.async_copy(
                        u_hbm.at[src_v.at[nj]], rows_v.at[b], gsems[b])
            return 0

        lax.fori_loop(0, s_steps // nb, blk, 0)
        for b in range(nb):
            pltpu.make_async_copy(
                rows_v.at[b], acc_sh.at[dst_v.at[s_steps - nb + b]],
                ssems[b]).wait()
        plsc.subcore_barrier()

        pltpu.sync_copy(acc_sh.at[pl.ds(s * rpt, rpt)], bounce_v)
        pltpu.sync_copy(bounce_v, out_hbm.at[c, pl.ds(s * rpt, rpt)])

    return propagate


def _dinv_block(cnt_ref):
    deg = cnt_ref[0, :, 0:1] + cnt_ref[1, :, 0:1] + 1.0
    return lax.rsqrt(deg)


def _t1_body(x_ref, w1_ref, cnt_ref, u1_ref):
    dinv = _dinv_block(cnt_ref)
    t = jnp.dot(x_ref[...], w1_ref[...], preferred_element_type=jnp.float32,
                precision=lax.Precision.HIGHEST)
    u1_ref[...] = t * dinv


def _t2_body(p_ref, u1_ref, cnt_ref, b1_ref, u2_ref):
    dinv = _dinv_block(cnt_ref)
    sfull = (p_ref[0] + p_ref[1] + u1_ref[...]) * dinv
    h = jnp.maximum(sfull + b1_ref[...], 0.0)
    u2_ref[...] = h * dinv


def _t3_body(q_ref, u2_ref, cnt_ref,
             wmu_ref, bmu_ref, wvar_ref, bvar_ref, mu_ref, sg_ref):
    dinv = _dinv_block(cnt_ref)
    z = (q_ref[0] + q_ref[1] + u2_ref[...]) * dinv
    mu_ref[...] = jnp.dot(z, wmu_ref[...], preferred_element_type=jnp.float32,
                          precision=lax.Precision.HIGHEST) + bmu_ref[...]
    sg_ref[...] = jnp.dot(z, wvar_ref[...], preferred_element_type=jnp.float32,
                          precision=lax.Precision.HIGHEST) + bvar_ref[...]


def kernel(x, edge_index, w1, b1, w_mu, b_mu, w_var, b_var):
    n, d_in = x.shape
    h_dim = w1.shape[1]
    d_out = w_mu.shape[1]
    e = edge_index.shape[1]

    n_pad = _cdiv(n + 1, NS * 8) * NS * 8
    e_rows = e // EB
    s_steps = _cdiv(_cdiv(e, NW * EB), 8) * 8
    pad_rows = s_steps - (e_rows - (NW - 1) * s_steps)

    pk = np.arange(pad_rows * EB, dtype=np.int32)
    pad_src = jnp.asarray((pk % n).reshape(pad_rows, EB))
    pad_dst = jnp.asarray((n + pk % (n_pad - n)).reshape(pad_rows, EB))

    src2d = edge_index[0].reshape(e_rows, EB)
    dst2d = edge_index[1].reshape(e_rows, EB)

    grid = (n // ROW_BLOCK,)
    row2 = lambda i: (i, 0)
    row3 = lambda i: (0, i, 0)
    full2 = lambda i: (0, 0)
    cnt_spec = pl.BlockSpec((NC, ROW_BLOCK, CW), row3)

    cnt = _make_degree_kernel(n_pad, s_steps, e_rows)(dst2d, pad_dst)

    u1 = pl.pallas_call(
        _t1_body,
        grid=grid,
        in_specs=[
            pl.BlockSpec((ROW_BLOCK, d_in), row2),
            pl.BlockSpec((d_in, h_dim), full2),
            cnt_spec,
        ],
        out_specs=pl.BlockSpec((ROW_BLOCK, h_dim), row2),
        out_shape=jax.ShapeDtypeStruct((n_pad, h_dim), jnp.float32),
    )(x, w1, cnt)

    prop = _make_propagate_kernel(n_pad, h_dim, s_steps, e_rows)
    p = prop(u1, src2d, dst2d, pad_src, pad_dst)

    u2 = pl.pallas_call(
        _t2_body,
        grid=grid,
        in_specs=[
            pl.BlockSpec((NC, ROW_BLOCK, h_dim), row3),
            pl.BlockSpec((ROW_BLOCK, h_dim), row2),
            cnt_spec,
            pl.BlockSpec((1, h_dim), full2),
        ],
        out_specs=pl.BlockSpec((ROW_BLOCK, h_dim), row2),
        out_shape=jax.ShapeDtypeStruct((n_pad, h_dim), jnp.float32),
    )(p, u1, cnt, b1.reshape(1, h_dim))

    q = prop(u2, src2d, dst2d, pad_src, pad_dst)

    mu, sg = pl.pallas_call(
        _t3_body,
        grid=grid,
        in_specs=[
            pl.BlockSpec((NC, ROW_BLOCK, h_dim), row3),
            pl.BlockSpec((ROW_BLOCK, h_dim), row2),
            cnt_spec,
            pl.BlockSpec((h_dim, d_out), full2),
            pl.BlockSpec((1, d_out), full2),
            pl.BlockSpec((h_dim, d_out), full2),
            pl.BlockSpec((1, d_out), full2),
        ],
        out_specs=[
            pl.BlockSpec((ROW_BLOCK, d_out), row2),
            pl.BlockSpec((ROW_BLOCK, d_out), row2),
        ],
        out_shape=[
            jax.ShapeDtypeStruct((n, d_out), jnp.float32),
            jax.ShapeDtypeStruct((n, d_out), jnp.float32),
        ],
    )(q, u2, cnt, w_mu, b_mu.reshape(1, d_out), w_var, b_var.reshape(1, d_out))

    return (mu, sg)


# speedup vs baseline: 1.1286x; 1.0620x over previous
import functools

import numpy as np
import jax
import jax.numpy as jnp
from jax import lax
from jax.experimental import pallas as pl
from jax.experimental.pallas import tpu as pltpu
from jax.experimental.pallas import tpu_sc as plsc

NC = 2
NS = 16
NW = NC * NS
LANES = 16
EB = 128
CW = 8
ROW_BLOCK = 2000


def _cdiv(a, b):
    return (a + b - 1) // b


def _mesh():
    return plsc.VectorSubcoreMesh(core_axis_name="c", subcore_axis_name="s")


_SC_PARAMS = pltpu.CompilerParams(use_tc_tiling_on_sc=False)


def _stage_idx(raw_hbm, pad_hbm, idx_v, wid, s_steps, e_rows, base):
    last = NW - 1
    r_real = e_rows - last * s_steps
    r_pad = s_steps - r_real

    @pl.when(wid < last)
    def _():
        pltpu.sync_copy(raw_hbm.at[pl.ds(base + wid * s_steps, s_steps)], idx_v)

    @pl.when(wid == last)
    def _():
        pltpu.sync_copy(raw_hbm.at[pl.ds(base + last * s_steps, r_real)],
                        idx_v.at[pl.ds(0, r_real)])
        pltpu.sync_copy(pad_hbm, idx_v.at[pl.ds(r_real, r_pad)])


def _make_degree_kernel(n_pad, s_steps, e_rows):
    rpt = n_pad // NS

    @functools.partial(
        pl.kernel,
        out_type=jax.ShapeDtypeStruct((NC, n_pad, CW), jnp.float32),
        mesh=_mesh(),
        compiler_params=_SC_PARAMS,
        scratch_types=[
            pltpu.VMEM((s_steps, EB), jnp.int32),
            pltpu.VMEM((EB, CW), jnp.float32),
            pltpu.VMEM((rpt, CW), jnp.float32),
            pltpu.VMEM_SHARED((n_pad, CW), jnp.float32),
            pltpu.SemaphoreType.DMA,
        ],
    )
    def degree(ei_hbm, pad_hbm, out_hbm, dst_v, ones_v, bounce_v, acc_sh, sem):
        c = lax.axis_index("c")
        s = lax.axis_index("s")
        wid = c * NS + s
        _stage_idx(ei_hbm, pad_hbm, dst_v, wid, s_steps, e_rows, e_rows)

        one = jnp.ones((LANES,), jnp.float32)
        zero = jnp.zeros((LANES,), jnp.float32)

        def fill_ones(i, _):
            ones_v[i, pl.ds(0, LANES)] = one
            return 0

        lax.fori_loop(0, EB, fill_ones, 0)

        def fill_zero(i, _):
            bounce_v[i, pl.ds(0, LANES)] = zero
            return 0

        lax.fori_loop(0, rpt, fill_zero, 0)
        pltpu.sync_copy(bounce_v, acc_sh.at[pl.ds(s * rpt, rpt)])
        plsc.subcore_barrier()

        K = 8

        def step(j2, _):
            cps = [pltpu.async_copy(ones_v, acc_sh.at[dst_v.at[j2 * K + b]],
                                    sem, add=True) for b in range(K)]
            for cp in cps:
                cp.wait()
            return 0

        lax.fori_loop(0, s_steps // K, step, 0)
        plsc.subcore_barrier()

        pltpu.sync_copy(acc_sh.at[pl.ds(s * rpt, rpt)], bounce_v)
        pltpu.sync_copy(bounce_v, out_hbm.at[c, pl.ds(s * rpt, rpt)])

    return degree


def _make_propagate_kernel(n_pad, width, s_steps, e_rows):
    rpt = n_pad // NS
    nb = 8
    assert s_steps % nb == 0

    @functools.partial(
        pl.kernel,
        out_type=jax.ShapeDtypeStruct((NC, n_pad, width), jnp.float32),
        mesh=_mesh(),
        compiler_params=_SC_PARAMS,
        scratch_types=[
            pltpu.VMEM((s_steps, EB), jnp.int32),
            pltpu.VMEM((s_steps, EB), jnp.int32),
            pltpu.VMEM((nb, EB, width), jnp.float32),
            pltpu.VMEM((rpt, width), jnp.float32),
            pltpu.VMEM_SHARED((n_pad, width), jnp.float32),
        ] + [pltpu.SemaphoreType.DMA] * (2 * nb),
    )
    def propagate(u_hbm, ei_hbm, psrc_hbm, pdst_hbm, out_hbm,
                  src_v, dst_v, rows_v, bounce_v, acc_sh, *sems):
        gsems = sems[:nb]
        ssems = sems[nb:]
        c = lax.axis_index("c")
        s = lax.axis_index("s")
        wid = c * NS + s
        _stage_idx(ei_hbm, psrc_hbm, src_v, wid, s_steps, e_rows, 0)
        _stage_idx(ei_hbm, pdst_hbm, dst_v, wid, s_steps, e_rows, e_rows)

        zero = jnp.zeros((LANES,), jnp.float32)

        def fill_zero(i, _):
            for k in range(width // LANES):
                bounce_v[i, pl.ds(k * LANES, LANES)] = zero
            return 0

        lax.fori_loop(0, rpt, fill_zero, 0)
        pltpu.sync_copy(bounce_v, acc_sh.at[pl.ds(s * rpt, rpt)])
        plsc.subcore_barrier()

        for b in range(nb):
            pltpu.async_copy(u_hbm.at[src_v.at[b]], rows_v.at[b], gsems[b])

        def blk(j2, _):
            base = j2 * nb
            for b in range(nb):
                j = base + b
                pltpu.make_async_copy(
                    u_hbm.at[src_v.at[j]], rows_v.at[b], gsems[b]).wait()
                pltpu.async_copy(rows_v.at[b], acc_sh.at[dst_v.at[j]],
                                 ssems[b], add=True)
            for b in range(nb):
                nj = base + nb + b

                @pl.when(nj < s_steps)
                def _():
                    pltpu.make_async_copy(
                        rows_v.at[b], acc_sh.at[dst_v.at[base + b]],
                        ssems[b]).wait()
                    pltpu.async_copy(
                        u_hbm.at[src_v.at[nj]], rows_v.at[b], gsems[b])
            return 0

        lax.fori_loop(0, s_steps // nb, blk, 0)
        for b in range(nb):
            pltpu.make_async_copy(
                rows_v.at[b], acc_sh.at[dst_v.at[s_steps - nb + b]],
                ssems[b]).wait()
        plsc.subcore_barrier()

        pltpu.sync_copy(acc_sh.at[pl.ds(s * rpt, rpt)], bounce_v)
        pltpu.sync_copy(bounce_v, out_hbm.at[c, pl.ds(s * rpt, rpt)])

    return propagate


def _dinv_block(cnt_ref):
    deg = cnt_ref[0, :, 0:1] + cnt_ref[1, :, 0:1] + 1.0
    return lax.rsqrt(deg)


def _t1_body(x_ref, w1_ref, cnt_ref, u1_ref):
    dinv = _dinv_block(cnt_ref)
    t = jnp.dot(x_ref[...], w1_ref[...], preferred_element_type=jnp.float32,
                precision=lax.Precision.HIGHEST)
    u1_ref[...] = t * dinv


def _t2_body(p_ref, u1_ref, cnt_ref, b1_ref, u2_ref):
    dinv = _dinv_block(cnt_ref)
    sfull = (p_ref[0] + p_ref[1] + u1_ref[...]) * dinv
    h = jnp.maximum(sfull + b1_ref[...], 0.0)
    u2_ref[...] = h * dinv


def _t3_body(q_ref, u2_ref, cnt_ref,
             wmu_ref, bmu_ref, wvar_ref, bvar_ref, mu_ref, sg_ref):
    dinv = _dinv_block(cnt_ref)
    z = (q_ref[0] + q_ref[1] + u2_ref[...]) * dinv
    mu_ref[...] = jnp.dot(z, wmu_ref[...], preferred_element_type=jnp.float32,
                          precision=lax.Precision.HIGHEST) + bmu_ref[...]
    sg_ref[...] = jnp.dot(z, wvar_ref[...], preferred_element_type=jnp.float32,
                          precision=lax.Precision.HIGHEST) + bvar_ref[...]


def kernel(x, edge_index, w1, b1, w_mu, b_mu, w_var, b_var):
    n, d_in = x.shape
    h_dim = w1.shape[1]
    d_out = w_mu.shape[1]
    e = edge_index.shape[1]

    n_pad = _cdiv(n + 1, NS * 8) * NS * 8
    e_rows = e // EB
    s_steps = _cdiv(_cdiv(e, NW * EB), 8) * 8
    pad_rows = s_steps - (e_rows - (NW - 1) * s_steps)

    pk = np.arange(pad_rows * EB, dtype=np.int32)
    pad_src = jnp.asarray((pk % n).reshape(pad_rows, EB))
    pad_dst = jnp.asarray((n + pk % (n_pad - n)).reshape(pad_rows, EB))

    ei2d = edge_index.reshape(2 * e_rows, EB)

    grid = (n // ROW_BLOCK,)
    row2 = lambda i: (i, 0)
    row3 = lambda i: (0, i, 0)
    full2 = lambda i: (0, 0)
    cnt_spec = pl.BlockSpec((NC, ROW_BLOCK, CW), row3)

    cnt = _make_degree_kernel(n_pad, s_steps, e_rows)(ei2d, pad_dst)

    u1 = pl.pallas_call(
        _t1_body,
        grid=grid,
        in_specs=[
            pl.BlockSpec((ROW_BLOCK, d_in), row2),
            pl.BlockSpec((d_in, h_dim), full2),
            cnt_spec,
        ],
        out_specs=pl.BlockSpec((ROW_BLOCK, h_dim), row2),
        out_shape=jax.ShapeDtypeStruct((n_pad, h_dim), jnp.float32),
    )(x, w1, cnt)

    prop = _make_propagate_kernel(n_pad, h_dim, s_steps, e_rows)
    p = prop(u1, ei2d, pad_src, pad_dst)

    u2 = pl.pallas_call(
        _t2_body,
        grid=grid,
        in_specs=[
            pl.BlockSpec((NC, ROW_BLOCK, h_dim), row3),
            pl.BlockSpec((ROW_BLOCK, h_dim), row2),
            cnt_spec,
            pl.BlockSpec((1, h_dim), full2),
        ],
        out_specs=pl.BlockSpec((ROW_BLOCK, h_dim), row2),
        out_shape=jax.ShapeDtypeStruct((n_pad, h_dim), jnp.float32),
    )(p, u1, cnt, b1.reshape(1, h_dim))

    q = prop(u2, ei2d, pad_src, pad_dst)

    mu, sg = pl.pallas_call(
        _t3_body,
        grid=grid,
        in_specs=[
            pl.BlockSpec((NC, ROW_BLOCK, h_dim), row3),
            pl.BlockSpec((ROW_BLOCK, h_dim), row2),
            cnt_spec,
            pl.BlockSpec((h_dim, d_out), full2),
            pl.BlockSpec((1, d_out), full2),
            pl.BlockSpec((h_dim, d_out), full2),
            pl.BlockSpec((1, d_out), full2),
        ],
        out_specs=[
            pl.BlockSpec((ROW_BLOCK, d_out), row2),
            pl.BlockSpec((ROW_BLOCK, d_out), row2),
        ],
        out_shape=[
            jax.ShapeDtypeStruct((n, d_out), jnp.float32),
            jax.ShapeDtypeStruct((n, d_out), jnp.float32),
        ],
    )(q, u2, cnt, w_mu, b_mu.reshape(1, d_out), w_var, b_var.reshape(1, d_out))

    return (mu, sg)
